# baseline (speedup 1.0000x reference)
import functools

import jax
import jax.numpy as jnp
from jax import lax
from jax.experimental import pallas as pl
from jax.experimental.pallas import tpu as pltpu
from jax.experimental.pallas import tpu_sc as plsc

NC = 2
NS = 16
L = 16
NW = NC * NS

NA, NB, NR, NCOL = 25, 128, 8, 128
NJ, NT, NE = 200, 128, 4
TT = 4
UNITS = NA * (NT // TT)
PER_W = UNITS // NW


def _make_embed():
    mesh = plsc.VectorSubcoreMesh(
        core_axis_name="c", subcore_axis_name="s", num_cores=NC, num_subcores=NS
    )

    @functools.partial(
        pl.kernel,
        out_type=jax.ShapeDtypeStruct((NJ, NT, NE, NCOL), jnp.float32),
        mesh=mesh,
        compiler_params=pltpu.CompilerParams(needs_layout_passes=False),
        scratch_types=[
            pltpu.VMEM((TT, NR, NCOL), jnp.int32),
            pltpu.VMEM((TT, NR, NCOL), jnp.int32),
            pltpu.VMEM((NR, TT, NE, NCOL), jnp.float32),
            pltpu.VMEM((NR, TT, NE, NCOL), jnp.float32),
            pltpu.VMEM((32,), jnp.float32),
            pltpu.SemaphoreType.DMA,
            pltpu.SemaphoreType.DMA,
            pltpu.SemaphoreType.DMA,
            pltpu.SemaphoreType.DMA,
        ],
    )
    def _embed(xP, tbl_hbm, oP, idx0, idx1, out0, out1, tbl_v, si0, si1, so0, so1):
        wid = lax.axis_index("s") * NC + lax.axis_index("c")
        pltpu.sync_copy(tbl_hbm, tbl_v)
        u0 = wid * PER_W

        def slabs(u):
            a = u // (NT // TT)
            tc = u % (NT // TT)
            t0 = tc * TT
            return (
                xP.at[a, pl.ds(t0, TT)],
                oP.at[pl.ds(a * NR, NR), pl.ds(t0, TT)],
            )

        def start_in(u, ib, sem):
            src, _ = slabs(u)
            pltpu.async_copy(src, ib, sem)

        def wait_in(u, ib, sem):
            src, _ = slabs(u)
            pltpu.make_async_copy(src, ib, sem).wait()

        def start_out(u, ob, sem):
            _, dst = slabs(u)
            pltpu.async_copy(ob, dst, sem)

        def wait_out(u, ob, sem):
            _, dst = slabs(u)
            pltpu.make_async_copy(ob, dst, sem).wait()

        def compute(idx_v, out_v):
            @plsc.parallel_loop(0, TT * NR, unroll=2)
            def rtbody(i):
                t = i // NR
                r = i % NR
                for ck in range(NCOL // L):
                    iv = idx_v[t, r, pl.ds(ck * L, L)]
                    for d in range(NE):
                        gi = iv if d == 0 else iv + (8 * d)
                        g = plsc.load_gather(tbl_v, [gi])
                        out_v[r, t, d, pl.ds(ck * L, L)] = g

        start_in(u0, idx0, si0)

        def pair(g, carry):
            ua = u0 + 2 * g
            start_in(ua + 1, idx1, si1)
            wait_in(ua, idx0, si0)
            pl.when(g > 0)(lambda: wait_out(ua - 2, out0, so0))
            compute(idx0, out0)
            start_out(ua, out0, so0)

            ub = ua + 1
            start_in(ub + 1, idx0, si0)
            wait_in(ub, idx1, si1)
            pl.when(g > 0)(lambda: wait_out(ub - 2, out1, so1))
            compute(idx1, out1)
            start_out(ub, out1, so1)
            return carry

        lax.fori_loop(0, (PER_W - 1) // 2, pair, 0)

        ul = u0 + PER_W - 1
        wait_in(ul, idx0, si0)
        wait_out(ul - 2, out0, so0)
        compute(idx0, out0)
        start_out(ul, out0, so0)
        wait_out(ul - 1, out1, so1)
        wait_out(ul, out0, so0)

    return _embed


_EMBED = None


def kernel(x, table):
    global _EMBED
    assert x.shape == (16384, 200) and table.shape == (7, 4)
    if _EMBED is None:
        _EMBED = _make_embed()
    xP = x.astype(jnp.int32).reshape(NB, NCOL, NA, NR).transpose(2, 0, 3, 1)
    tcol = jnp.pad(table.T, ((0, 0), (0, 1))).reshape(32)
    oP = _EMBED(xP, tcol)
    return oP.transpose(1, 3, 0, 2).reshape(16384, 200, 4)


# speedup vs baseline: 1.0093x; 1.0093x over previous
import functools

import jax
import jax.numpy as jnp
from jax import lax
from jax.experimental import pallas as pl
from jax.experimental.pallas import tpu as pltpu
from jax.experimental.pallas import tpu_sc as plsc

NC = 2
NS = 16
L = 16
NW = NC * NS

NA, NB, NR, NCOL = 25, 128, 8, 128
NJ, NT, NE = 200, 128, 4
TT = 32
UNITS = NJ * (NT // TT)
PER_W = UNITS // NW


def _make_embed():
    mesh = plsc.VectorSubcoreMesh(
        core_axis_name="c", subcore_axis_name="s", num_cores=NC, num_subcores=NS
    )

    @functools.partial(
        pl.kernel,
        out_type=jax.ShapeDtypeStruct((NJ, NT, NE, NCOL), jnp.float32),
        mesh=mesh,
        compiler_params=pltpu.CompilerParams(needs_layout_passes=False),
        scratch_types=[
            pltpu.VMEM((TT, NCOL), jnp.int32),
            pltpu.VMEM((TT, NCOL), jnp.int32),
            pltpu.VMEM((TT, NE, NCOL), jnp.float32),
            pltpu.VMEM((TT, NE, NCOL), jnp.float32),
            pltpu.VMEM((32,), jnp.float32),
            pltpu.SemaphoreType.DMA,
            pltpu.SemaphoreType.DMA,
            pltpu.SemaphoreType.DMA,
            pltpu.SemaphoreType.DMA,
        ],
    )
    def _embed(xP, tbl_hbm, oP, idx0, idx1, out0, out1, tbl_v, si0, si1, so0, so1):
        wid = lax.axis_index("s") * NC + lax.axis_index("c")
        pltpu.sync_copy(tbl_hbm, tbl_v)
        u0 = wid * PER_W

        def slabs(u):
            j = u // (NT // TT)
            tc = u % (NT // TT)
            a = j // NR
            r = j % NR
            t0 = tc * TT
            return (
                xP.at[a, pl.ds(t0, TT), r],
                oP.at[j, pl.ds(t0, TT)],
            )

        def start_in(u, ib, sem):
            src, _ = slabs(u)
            pltpu.async_copy(src, ib, sem)

        def wait_in(u, ib, sem):
            src, _ = slabs(u)
            pltpu.make_async_copy(src, ib, sem).wait()

        def start_out(u, ob, sem):
            _, dst = slabs(u)
            pltpu.async_copy(ob, dst, sem)

        def wait_out(u, ob, sem):
            _, dst = slabs(u)
            pltpu.make_async_copy(ob, dst, sem).wait()

        def compute(idx_v, out_v):
            @plsc.parallel_loop(0, TT, unroll=4)
            def tbody(t):
                for ck in range(NCOL // L):
                    iv = idx_v[t, pl.ds(ck * L, L)]
                    for d in range(NE):
                        gi = iv if d == 0 else iv + (8 * d)
                        g = plsc.load_gather(tbl_v, [gi])
                        out_v[t, d, pl.ds(ck * L, L)] = g

        start_in(u0, idx0, si0)

        def pair(g, carry):
            ua = u0 + 2 * g
            start_in(ua + 1, idx1, si1)
            wait_in(ua, idx0, si0)
            pl.when(g > 0)(lambda: wait_out(ua - 2, out0, so0))
            compute(idx0, out0)
            start_out(ua, out0, so0)

            ub = ua + 1
            start_in(ub + 1, idx0, si0)
            wait_in(ub, idx1, si1)
            pl.when(g > 0)(lambda: wait_out(ub - 2, out1, so1))
            compute(idx1, out1)
            start_out(ub, out1, so1)
            return carry

        lax.fori_loop(0, (PER_W - 1) // 2, pair, 0)

        ul = u0 + PER_W - 1
        wait_in(ul, idx0, si0)
        wait_out(ul - 2, out0, so0)
        compute(idx0, out0)
        start_out(ul, out0, so0)
        wait_out(ul - 1, out1, so1)
        wait_out(ul, out0, so0)

    return _embed


_EMBED = None


def kernel(x, table):
    global _EMBED
    assert x.shape == (16384, 200) and table.shape == (7, 4)
    if _EMBED is None:
        _EMBED = _make_embed()
    xP = x.astype(jnp.int32).reshape(NB, NCOL, NA, NR).transpose(2, 0, 3, 1)
    tcol = jnp.pad(table.T, ((0, 0), (0, 1))).reshape(32)
    oP = _EMBED(xP, tcol)
    return oP.transpose(1, 3, 0, 2).reshape(16384, 200, 4)
